# baseline (speedup 1.0000x reference)
import functools

import jax
import jax.numpy as jnp
from jax import lax
from jax.experimental import pallas as pl
from jax.experimental.pallas import tpu as pltpu
from jax.experimental.pallas import tpu_sc as plsc

_IN = 128
_OUT = 128
_H = 8
_DH = 16
_FFN = 512
_DE = 16
_NW = 10000
_NS = 10000
_E = 320000

_EXT = 144
_NEG = -1e30

_PREA_BLK = 2000
_PREB_BLK = 8000
_POST_BLK = 2000

_CHUNK = 80
_N_WORKERS = 32
_EPW = _E // _N_WORKERS
_NCHUNK = _EPW // _CHUNK

_HP = jax.lax.Precision.HIGHEST


def _prea_body(w_ref, wcat_ref, wa1p_ref, zext_ref, zamax_ref):
    wb = w_ref[...]
    z2 = jnp.dot(wb, wcat_ref[...], precision=_HP)
    za16 = jnp.dot(wb, wa1p_ref[...], precision=_HP)
    pad_bias = jnp.where(lax.broadcasted_iota(jnp.int32, (1, 16), 1) >= _H,
                         jnp.float32(_NEG), jnp.float32(0.0))
    za16 = za16 + pad_bias
    zext_ref[...] = jnp.concatenate([z2, za16], axis=1)
    bm = jnp.max(za16, axis=0, keepdims=True)
    prev = jnp.where(pl.program_id(0) == 0,
                     jnp.full((1, 16), _NEG, jnp.float32), zamax_ref[...])
    zamax_ref[...] = jnp.maximum(prev, bm)


def _preb_body(ea_ref, wa2p_ref, bap_ref, e16_ref, eamax_ref):
    eb = ea_ref[...]
    e16 = jnp.dot(eb, wa2p_ref[...], precision=_HP) + bap_ref[...]
    e16_ref[...] = e16
    bm = jnp.max(e16, axis=0, keepdims=True)
    prev = jnp.where(pl.program_id(0) == 0,
                     jnp.full((1, 16), _NEG, jnp.float32), eamax_ref[...])
    eamax_ref[...] = jnp.maximum(prev, bm)


def _sc_edge_body(zext_hbm, src_hbm, dst_hbm, ea16_hbm, zam_hbm, eam_hbm,
                  zeros_hbm, out_hbm,
                  src_v0, src_v1, dst_v0, dst_v1, ea_v0, ea_v1,
                  rows_v0, rows_v1, m16_v, acc_sh,
                  isem0, isem1, gsem0, gsem1):
    c = lax.axis_index("c")
    s = lax.axis_index("s")
    wid = c * 16 + s
    wbase = wid * _EPW
    src_v = (src_v0, src_v1)
    dst_v = (dst_v0, dst_v1)
    ea_v = (ea_v0, ea_v1)
    rows_v = (rows_v0, rows_v1)
    isem = (isem0, isem1)
    gsem = (gsem0, gsem1)

    if True:
        @pl.when(s == 0)
        def _():
            pltpu.sync_copy(zeros_hbm, acc_sh)
        plsc.subcore_barrier()

        pltpu.sync_copy(zam_hbm, m16_v)
        zam = m16_v[...]
        pltpu.sync_copy(eam_hbm, m16_v)
        m = zam + m16_v[...]
        c16 = jnp.maximum(m, 0.01 * m)

        def idx_descs(j, k):
            base = wbase + j * _CHUNK
            return (
                pltpu.make_async_copy(src_hbm.at[pl.ds(base, _CHUNK)],
                                      src_v[k], isem[k]),
                pltpu.make_async_copy(dst_hbm.at[pl.ds(base, _CHUNK)],
                                      dst_v[k], isem[k]),
                pltpu.make_async_copy(ea16_hbm.at[pl.ds(base, _CHUNK)],
                                      ea_v[k], isem[k]),
            )

        def gather_desc(k):
            return pltpu.make_async_copy(zext_hbm.at[src_v[k]],
                                         rows_v[k], gsem[k])

        def compute(k):
            rv = rows_v[k]
            ev = ea_v[k]

            def edge_body(i, carry2):
                za = rv[i, pl.ds(128, 16)]
                v = za + ev[i, :]
                lr = jnp.maximum(v, 0.01 * v)
                p = jnp.exp(lr - c16)
                loads = [rv[i, pl.ds(h * 16, 16)] for h in range(_H)]
                rv[i, pl.ds(128, 16)] = p
                for h in range(_H):
                    rv[i, pl.ds(h * 16, 16)] = loads[h] * p[h]
                return carry2

            lax.fori_loop(0, _CHUNK, edge_body, 0, unroll=8)

        def do_chunk(j, k):
            k1 = 1 - k
            gather_desc(k).wait()

            @pl.when(j + 1 < _NCHUNK)
            def _():
                for d in idx_descs(j + 1, k1):
                    d.wait()
                gather_desc(k1).start()

            compute(k)
            pltpu.sync_copy(rows_v[k], acc_sh.at[dst_v[k]], add=True)

            @pl.when(j + 2 < _NCHUNK)
            def _():
                for d in idx_descs(j + 2, k):
                    d.start()

        for d in idx_descs(0, 0):
            d.start()
        for d in idx_descs(1, 1):
            d.start()
        for d in idx_descs(0, 0):
            d.wait()
        gather_desc(0).start()

        def pair_body(jj, carry):
            do_chunk(2 * jj, 0)
            do_chunk(2 * jj + 1, 1)
            return carry

        lax.fori_loop(0, _NCHUNK // 2, pair_body, 0)
        if _NCHUNK % 2:
            do_chunk(_NCHUNK - 1, 0)
        plsc.subcore_barrier()

        r0 = s * 624
        pltpu.sync_copy(acc_sh.at[pl.ds(r0, 624)],
                        out_hbm.at[c].at[pl.ds(r0, 624)])

        @pl.when(s == 15)
        def _():
            pltpu.sync_copy(acc_sh.at[pl.ds(9984, 16)],
                            out_hbm.at[c].at[pl.ds(9984, 16)])


def _post_body(p0_ref, p1_ref, s_ref, g_ref, b_ref, w1_ref, b1_ref,
               w2_ref, b2_ref, o_ref):
    P = p0_ref[...] + p1_ref[...]
    hacc = P[:, :128]
    den = P[:, 128:136]
    recip = 1.0 / (den + 1e-10)
    ii = lax.broadcasted_iota(jnp.int32, (_H, 128), 0)
    jj = lax.broadcasted_iota(jnp.int32, (_H, 128), 1)
    expand = jnp.where(ii == jj // 16, 1.0, 0.0).astype(jnp.float32)
    hsum = hacc * jnp.dot(recip, expand, precision=_HP)
    h = jnp.where(hsum > 0, hsum, jnp.exp(jnp.minimum(hsum, 0.0)) - 1.0)
    h = h + s_ref[...]
    mu = jnp.mean(h, axis=-1, keepdims=True)
    var = jnp.mean((h - mu) ** 2, axis=-1, keepdims=True)
    hn = (h - mu) * lax.rsqrt(var + 1e-6) * g_ref[...] + b_ref[...]
    pre = jnp.dot(hn, w1_ref[...], precision=_HP) + b1_ref[...]
    inter = 0.5 * pre * (1.0 + lax.erf(pre * 0.7071067811865476))
    o_ref[...] = jnp.dot(inter, w2_ref[...], precision=_HP) + b2_ref[...] + h


def _full(shape):
    return pl.BlockSpec(shape, lambda i: (0, 0))


def _prea(w, wcat, wa1p):
    grid = _NW // _PREA_BLK
    return pl.pallas_call(
        _prea_body,
        grid=(grid,),
        in_specs=[
            pl.BlockSpec((_PREA_BLK, _IN), lambda i: (i, 0)),
            _full((_IN, _OUT)),
            _full((_IN, 16)),
        ],
        out_specs=[
            pl.BlockSpec((_PREA_BLK, _EXT), lambda i: (i, 0)),
            _full((1, 16)),
        ],
        out_shape=[
            jax.ShapeDtypeStruct((_NW, _EXT), jnp.float32),
            jax.ShapeDtypeStruct((1, 16), jnp.float32),
        ],
    )(w, wcat, wa1p)


def _preb(edge_attr, wa2p, bap):
    grid = _E // _PREB_BLK
    return pl.pallas_call(
        _preb_body,
        grid=(grid,),
        in_specs=[
            pl.BlockSpec((_PREB_BLK, _DE), lambda i: (i, 0)),
            _full((_DE, 16)),
            _full((1, 16)),
        ],
        out_specs=[
            pl.BlockSpec((_PREB_BLK, 16), lambda i: (i, 0)),
            _full((1, 16)),
        ],
        out_shape=[
            jax.ShapeDtypeStruct((_E, 16), jnp.float32),
            jax.ShapeDtypeStruct((1, 16), jnp.float32),
        ],
    )(edge_attr, wa2p, bap)


@functools.cache
def _sc_edge_kernel():
    @functools.partial(
        pl.kernel,
        mesh=plsc.VectorSubcoreMesh(core_axis_name="c", subcore_axis_name="s"),
        out_type=jax.ShapeDtypeStruct((2, _NS, _EXT), jnp.float32),
        compiler_params=pltpu.CompilerParams(use_tc_tiling_on_sc=False),
        scratch_types=[
            pltpu.VMEM((_CHUNK,), jnp.int32),
            pltpu.VMEM((_CHUNK,), jnp.int32),
            pltpu.VMEM((_CHUNK,), jnp.int32),
            pltpu.VMEM((_CHUNK,), jnp.int32),
            pltpu.VMEM((_CHUNK, 16), jnp.float32),
            pltpu.VMEM((_CHUNK, 16), jnp.float32),
            pltpu.VMEM((_CHUNK, _EXT), jnp.float32),
            pltpu.VMEM((_CHUNK, _EXT), jnp.float32),
            pltpu.VMEM((16,), jnp.float32),
            pltpu.VMEM_SHARED((_NS, _EXT), jnp.float32),
            pltpu.SemaphoreType.DMA,
            pltpu.SemaphoreType.DMA,
            pltpu.SemaphoreType.DMA,
            pltpu.SemaphoreType.DMA,
        ],
    )
    def _sc_edge(*refs):
        _sc_edge_body(*refs)

    return _sc_edge


def _post(p0, p1, s, ln_g, ln_b, w1, b1, w2, b2):
    grid = _NS // _POST_BLK
    return pl.pallas_call(
        _post_body,
        grid=(grid,),
        in_specs=[
            pl.BlockSpec((_POST_BLK, _EXT), lambda i: (i, 0)),
            pl.BlockSpec((_POST_BLK, _EXT), lambda i: (i, 0)),
            pl.BlockSpec((_POST_BLK, _OUT), lambda i: (i, 0)),
            _full((1, _OUT)),
            _full((1, _OUT)),
            _full((_OUT, _FFN)),
            _full((1, _FFN)),
            _full((_FFN, _OUT)),
            _full((1, _OUT)),
        ],
        out_specs=pl.BlockSpec((_POST_BLK, _OUT), lambda i: (i, 0)),
        out_shape=jax.ShapeDtypeStruct((_NS, _OUT), jnp.float32),
    )(p0, p1, s, ln_g, ln_b, w1, b1, w2, b2)


def kernel(w, s, edge_index, edge_attr, W_fc, attn_a, W_feat, b_feat,
           ln_g, ln_b, w1, b1, w2, b2):
    src = edge_index[0]
    dst = edge_index[1]
    a1 = attn_a[:, :_DH]
    a2 = attn_a[:, _DH:]
    wcat = jnp.transpose(W_fc, (1, 0, 2)).reshape(_IN, _OUT)
    wa1p = jnp.pad(jnp.einsum('hdo,ho->dh', W_fc, a1), ((0, 0), (0, 8)))
    wa2p = jnp.pad(jnp.einsum('hdo,ho->dh', W_feat, a2), ((0, 0), (0, 8)))
    bap = jnp.pad(jnp.einsum('ho,ho->h', b_feat, a2), (0, 8)).reshape(1, 16)

    z2ext, zamax = _prea(w, wcat, wa1p)
    ea16, eamax = _preb(edge_attr, wa2p, bap)

    zeros = jnp.zeros((_NS, _EXT), jnp.float32)
    partials = _sc_edge_kernel()(z2ext, src, dst, ea16,
                                 zamax.reshape(16), eamax.reshape(16), zeros)

    return _post(partials[0], partials[1], s,
                 ln_g.reshape(1, _OUT), ln_b.reshape(1, _OUT),
                 w1, b1.reshape(1, _FFN), w2, b2.reshape(1, _OUT))


# speedup vs baseline: 1.0123x; 1.0123x over previous
import functools

import jax
import jax.numpy as jnp
from jax import lax
from jax.experimental import pallas as pl
from jax.experimental.pallas import tpu as pltpu
from jax.experimental.pallas import tpu_sc as plsc

_IN = 128
_OUT = 128
_H = 8
_DH = 16
_FFN = 512
_DE = 16
_NW = 10000
_NS = 10000
_E = 320000

_EXT = 144
_NEG = -1e30

_PREA_BLK = 2000
_PREB_BLK = 8000
_POST_BLK = 2000

_CHUNK = 80
_N_WORKERS = 32
_EPW = _E // _N_WORKERS
_NCHUNK = _EPW // _CHUNK

_HP = jax.lax.Precision.HIGHEST


def _prea_body(w_ref, wcat_ref, wa1p_ref, zext_ref, zamax_ref):
    wb = w_ref[...]
    z2 = jnp.dot(wb, wcat_ref[...], precision=_HP)
    za16 = jnp.dot(wb, wa1p_ref[...], precision=_HP)
    pad_bias = jnp.where(lax.broadcasted_iota(jnp.int32, (1, 16), 1) >= _H,
                         jnp.float32(_NEG), jnp.float32(0.0))
    za16 = za16 + pad_bias
    zext_ref[...] = jnp.concatenate([z2, za16], axis=1)
    bm = jnp.max(za16, axis=0, keepdims=True)
    prev = jnp.where(pl.program_id(0) == 0,
                     jnp.full((1, 16), _NEG, jnp.float32), zamax_ref[...])
    zamax_ref[...] = jnp.maximum(prev, bm)


def _preb_body(ea_ref, wa2p_ref, bap_ref, e16_ref, eamax_ref):
    eb = ea_ref[...]
    e16 = jnp.dot(eb, wa2p_ref[...], precision=_HP) + bap_ref[...]
    e16_ref[...] = e16
    bm = jnp.max(e16, axis=0, keepdims=True)
    prev = jnp.where(pl.program_id(0) == 0,
                     jnp.full((1, 16), _NEG, jnp.float32), eamax_ref[...])
    eamax_ref[...] = jnp.maximum(prev, bm)


def _sc_edge_body(zext_hbm, src_hbm, dst_hbm, ea16_hbm, zam_hbm, eam_hbm,
                  zeros_hbm, out_hbm,
                  src_v0, src_v1, dst_v0, dst_v1, ea_v0, ea_v1,
                  rows_v0, rows_v1, m16_v, acc_sh,
                  isem0, isem1, gsem0, gsem1):
    c = lax.axis_index("c")
    s = lax.axis_index("s")
    wid = c * 16 + s
    wbase = wid * _EPW
    src_v = (src_v0, src_v1)
    dst_v = (dst_v0, dst_v1)
    ea_v = (ea_v0, ea_v1)
    rows_v = (rows_v0, rows_v1)
    isem = (isem0, isem1)
    gsem = (gsem0, gsem1)

    if True:
        @pl.when(s == 0)
        def _():
            pltpu.sync_copy(zeros_hbm, acc_sh)
        plsc.subcore_barrier()

        pltpu.sync_copy(zam_hbm, m16_v)
        zam = m16_v[...]
        pltpu.sync_copy(eam_hbm, m16_v)
        m = zam + m16_v[...]
        c16 = jnp.maximum(m, 0.01 * m)

        def idx_descs(j, k):
            base = wbase + j * _CHUNK
            return (
                pltpu.make_async_copy(src_hbm.at[pl.ds(base, _CHUNK)],
                                      src_v[k], isem[k]),
                pltpu.make_async_copy(dst_hbm.at[pl.ds(base, _CHUNK)],
                                      dst_v[k], isem[k]),
                pltpu.make_async_copy(ea16_hbm.at[pl.ds(base, _CHUNK)],
                                      ea_v[k], isem[k]),
            )

        def gather_desc(k):
            return pltpu.make_async_copy(zext_hbm.at[src_v[k]],
                                         rows_v[k], gsem[k])

        def compute(k):
            rv = rows_v[k]
            ev = ea_v[k]

            def p_body(i, carry2):
                za = rv[i, pl.ds(128, 16)]
                v = za + ev[i, :]
                lr = jnp.maximum(v, 0.01 * v)
                rv[i, pl.ds(128, 16)] = jnp.exp(lr - c16)
                return carry2

            lax.fori_loop(0, _CHUNK, p_body, 0, unroll=8)

            def scale_body(i, carry2):
                p = rv[i, pl.ds(128, 16)]
                loads = [rv[i, pl.ds(h * 16, 16)] for h in range(_H)]
                for h in range(_H):
                    rv[i, pl.ds(h * 16, 16)] = loads[h] * p[h]
                return carry2

            lax.fori_loop(0, _CHUNK, scale_body, 0, unroll=4)

        def do_chunk(j, k):
            k1 = 1 - k
            gather_desc(k).wait()

            @pl.when(j + 1 < _NCHUNK)
            def _():
                for d in idx_descs(j + 1, k1):
                    d.wait()
                gather_desc(k1).start()

            compute(k)
            pltpu.sync_copy(rows_v[k], acc_sh.at[dst_v[k]], add=True)

            @pl.when(j + 2 < _NCHUNK)
            def _():
                for d in idx_descs(j + 2, k):
                    d.start()

        for d in idx_descs(0, 0):
            d.start()
        for d in idx_descs(1, 1):
            d.start()
        for d in idx_descs(0, 0):
            d.wait()
        gather_desc(0).start()

        def pair_body(jj, carry):
            do_chunk(2 * jj, 0)
            do_chunk(2 * jj + 1, 1)
            return carry

        lax.fori_loop(0, _NCHUNK // 2, pair_body, 0)
        if _NCHUNK % 2:
            do_chunk(_NCHUNK - 1, 0)
        plsc.subcore_barrier()

        r0 = s * 624
        pltpu.sync_copy(acc_sh.at[pl.ds(r0, 624)],
                        out_hbm.at[c].at[pl.ds(r0, 624)])

        @pl.when(s == 15)
        def _():
            pltpu.sync_copy(acc_sh.at[pl.ds(9984, 16)],
                            out_hbm.at[c].at[pl.ds(9984, 16)])


def _post_body(p0_ref, p1_ref, s_ref, g_ref, b_ref, w1_ref, b1_ref,
               w2_ref, b2_ref, o_ref):
    P = p0_ref[...] + p1_ref[...]
    hacc = P[:, :128]
    den = P[:, 128:136]
    recip = 1.0 / (den + 1e-10)
    ii = lax.broadcasted_iota(jnp.int32, (_H, 128), 0)
    jj = lax.broadcasted_iota(jnp.int32, (_H, 128), 1)
    expand = jnp.where(ii == jj // 16, 1.0, 0.0).astype(jnp.float32)
    hsum = hacc * jnp.dot(recip, expand, precision=_HP)
    h = jnp.where(hsum > 0, hsum, jnp.exp(jnp.minimum(hsum, 0.0)) - 1.0)
    h = h + s_ref[...]
    mu = jnp.mean(h, axis=-1, keepdims=True)
    var = jnp.mean((h - mu) ** 2, axis=-1, keepdims=True)
    hn = (h - mu) * lax.rsqrt(var + 1e-6) * g_ref[...] + b_ref[...]
    pre = jnp.dot(hn, w1_ref[...], precision=_HP) + b1_ref[...]
    inter = 0.5 * pre * (1.0 + lax.erf(pre * 0.7071067811865476))
    o_ref[...] = jnp.dot(inter, w2_ref[...], precision=_HP) + b2_ref[...] + h


def _full(shape):
    return pl.BlockSpec(shape, lambda i: (0, 0))


def _prea(w, wcat, wa1p):
    grid = _NW // _PREA_BLK
    return pl.pallas_call(
        _prea_body,
        grid=(grid,),
        in_specs=[
            pl.BlockSpec((_PREA_BLK, _IN), lambda i: (i, 0)),
            _full((_IN, _OUT)),
            _full((_IN, 16)),
        ],
        out_specs=[
            pl.BlockSpec((_PREA_BLK, _EXT), lambda i: (i, 0)),
            _full((1, 16)),
        ],
        out_shape=[
            jax.ShapeDtypeStruct((_NW, _EXT), jnp.float32),
            jax.ShapeDtypeStruct((1, 16), jnp.float32),
        ],
    )(w, wcat, wa1p)


def _preb(edge_attr, wa2p, bap):
    grid = _E // _PREB_BLK
    return pl.pallas_call(
        _preb_body,
        grid=(grid,),
        in_specs=[
            pl.BlockSpec((_PREB_BLK, _DE), lambda i: (i, 0)),
            _full((_DE, 16)),
            _full((1, 16)),
        ],
        out_specs=[
            pl.BlockSpec((_PREB_BLK, 16), lambda i: (i, 0)),
            _full((1, 16)),
        ],
        out_shape=[
            jax.ShapeDtypeStruct((_E, 16), jnp.float32),
            jax.ShapeDtypeStruct((1, 16), jnp.float32),
        ],
    )(edge_attr, wa2p, bap)


@functools.cache
def _sc_edge_kernel():
    @functools.partial(
        pl.kernel,
        mesh=plsc.VectorSubcoreMesh(core_axis_name="c", subcore_axis_name="s"),
        out_type=jax.ShapeDtypeStruct((2, _NS, _EXT), jnp.float32),
        compiler_params=pltpu.CompilerParams(use_tc_tiling_on_sc=False),
        scratch_types=[
            pltpu.VMEM((_CHUNK,), jnp.int32),
            pltpu.VMEM((_CHUNK,), jnp.int32),
            pltpu.VMEM((_CHUNK,), jnp.int32),
            pltpu.VMEM((_CHUNK,), jnp.int32),
            pltpu.VMEM((_CHUNK, 16), jnp.float32),
            pltpu.VMEM((_CHUNK, 16), jnp.float32),
            pltpu.VMEM((_CHUNK, _EXT), jnp.float32),
            pltpu.VMEM((_CHUNK, _EXT), jnp.float32),
            pltpu.VMEM((16,), jnp.float32),
            pltpu.VMEM_SHARED((_NS, _EXT), jnp.float32),
            pltpu.SemaphoreType.DMA,
            pltpu.SemaphoreType.DMA,
            pltpu.SemaphoreType.DMA,
            pltpu.SemaphoreType.DMA,
        ],
    )
    def _sc_edge(*refs):
        _sc_edge_body(*refs)

    return _sc_edge


def _post(p0, p1, s, ln_g, ln_b, w1, b1, w2, b2):
    grid = _NS // _POST_BLK
    return pl.pallas_call(
        _post_body,
        grid=(grid,),
        in_specs=[
            pl.BlockSpec((_POST_BLK, _EXT), lambda i: (i, 0)),
            pl.BlockSpec((_POST_BLK, _EXT), lambda i: (i, 0)),
            pl.BlockSpec((_POST_BLK, _OUT), lambda i: (i, 0)),
            _full((1, _OUT)),
            _full((1, _OUT)),
            _full((_OUT, _FFN)),
            _full((1, _FFN)),
            _full((_FFN, _OUT)),
            _full((1, _OUT)),
        ],
        out_specs=pl.BlockSpec((_POST_BLK, _OUT), lambda i: (i, 0)),
        out_shape=jax.ShapeDtypeStruct((_NS, _OUT), jnp.float32),
    )(p0, p1, s, ln_g, ln_b, w1, b1, w2, b2)


def kernel(w, s, edge_index, edge_attr, W_fc, attn_a, W_feat, b_feat,
           ln_g, ln_b, w1, b1, w2, b2):
    src = edge_index[0]
    dst = edge_index[1]
    a1 = attn_a[:, :_DH]
    a2 = attn_a[:, _DH:]
    wcat = jnp.transpose(W_fc, (1, 0, 2)).reshape(_IN, _OUT)
    wa1p = jnp.pad(jnp.einsum('hdo,ho->dh', W_fc, a1), ((0, 0), (0, 8)))
    wa2p = jnp.pad(jnp.einsum('hdo,ho->dh', W_feat, a2), ((0, 0), (0, 8)))
    bap = jnp.pad(jnp.einsum('ho,ho->h', b_feat, a2), (0, 8)).reshape(1, 16)

    z2ext, zamax = _prea(w, wcat, wa1p)
    ea16, eamax = _preb(edge_attr, wa2p, bap)

    zeros = jnp.zeros((_NS, _EXT), jnp.float32)
    partials = _sc_edge_kernel()(z2ext, src, dst, ea16,
                                 zamax.reshape(16), eamax.reshape(16), zeros)

    return _post(partials[0], partials[1], s,
                 ln_g.reshape(1, _OUT), ln_b.reshape(1, _OUT),
                 w1, b1.reshape(1, _FFN), w2, b2.reshape(1, _OUT))


# speedup vs baseline: 1.3285x; 1.3124x over previous
import functools

import jax
import jax.numpy as jnp
from jax import lax
from jax.experimental import pallas as pl
from jax.experimental.pallas import tpu as pltpu
from jax.experimental.pallas import tpu_sc as plsc

_IN = 128
_OUT = 128
_H = 8
_DH = 16
_FFN = 512
_DE = 16
_NW = 10000
_NS = 10000
_E = 320000

_EXT = 144
_NEG = -1e30

_PREA_BLK = 2000
_PREB_BLK = 8000
_POST_BLK = 2000

_CHUNK = 80
_N_WORKERS = 32
_EPW = _E // _N_WORKERS
_NCHUNK = _EPW // _CHUNK

_HP = jax.lax.Precision.HIGHEST


def _prea_body(w_ref, wcat_ref, wa1p_ref, zext_ref, zamax_ref):
    wb = w_ref[...]
    z2 = jnp.dot(wb, wcat_ref[...], precision=_HP)
    za16 = jnp.dot(wb, wa1p_ref[...], precision=_HP)
    pad_bias = jnp.where(lax.broadcasted_iota(jnp.int32, (1, 16), 1) >= _H,
                         jnp.float32(_NEG), jnp.float32(0.0))
    za16 = za16 + pad_bias
    zext_ref[...] = jnp.concatenate([z2, za16], axis=1)
    bm = jnp.max(za16, axis=0, keepdims=True)
    prev = jnp.where(pl.program_id(0) == 0,
                     jnp.full((1, 16), _NEG, jnp.float32), zamax_ref[...])
    zamax_ref[...] = jnp.maximum(prev, bm)


def _preb_body(ea_ref, wb_ref, ba_ref, e_ref, eamax_ref):
    eb = ea_ref[...]
    e = jnp.dot(eb, wb_ref[...], precision=_HP) + ba_ref[...]
    e_ref[...] = e
    bm = jnp.max(e, axis=0, keepdims=True)
    prev = jnp.where(pl.program_id(0) == 0,
                     jnp.full((1, 128), _NEG, jnp.float32), eamax_ref[...])
    eamax_ref[...] = jnp.maximum(prev, bm)


def _sc_edge_body(zext_hbm, src_hbm, dst_hbm, eapk_hbm, zam_hbm, eam_hbm,
                  zeros_hbm, out_hbm,
                  src_v0, src_v1, dst_v0, dst_v1, ea_v0, ea_v1,
                  rows_v0, rows_v1, m16_v, m128_v, acc_sh,
                  isem0, isem1, gsem0, gsem1):
    c = lax.axis_index("c")
    s = lax.axis_index("s")
    wid = c * 16 + s
    wbase = wid * _EPW
    src_v = (src_v0, src_v1)
    dst_v = (dst_v0, dst_v1)
    ea_v = (ea_v0, ea_v1)
    rows_v = (rows_v0, rows_v1)
    isem = (isem0, isem1)
    gsem = (gsem0, gsem1)

    if True:
        @pl.when(s == 0)
        def _():
            pltpu.sync_copy(zeros_hbm, acc_sh)
        plsc.subcore_barrier()

        pltpu.sync_copy(zam_hbm, m16_v)
        zam = m16_v[...]
        pltpu.sync_copy(eam_hbm, m128_v)
        em = m128_v[pl.ds(0, 16)]
        for jj in range(1, 8):
            em = jnp.maximum(em, m128_v[pl.ds(jj * 16, 16)])
        m = zam + em
        c16 = jnp.maximum(m, 0.01 * m)

        def idx_descs(j, k):
            base = wbase + j * _CHUNK
            return (
                pltpu.make_async_copy(src_hbm.at[pl.ds(base, _CHUNK)],
                                      src_v[k], isem[k]),
                pltpu.make_async_copy(dst_hbm.at[pl.ds(base, _CHUNK)],
                                      dst_v[k], isem[k]),
                pltpu.make_async_copy(
                    eapk_hbm.at[pl.ds(base // 8, _CHUNK // 8)],
                    ea_v[k], isem[k]),
            )

        def gather_desc(k):
            return pltpu.make_async_copy(zext_hbm.at[src_v[k]],
                                         rows_v[k], gsem[k])

        def compute(k):
            rv = rows_v[k]
            ev = ea_v[k]

            def p_body(r, carry2):
                for j in range(8):
                    i = r * 8 + j
                    za = rv[i, pl.ds(128, 16)]
                    v = za + ev[r, pl.ds(j * 16, 16)]
                    lr = jnp.maximum(v, 0.01 * v)
                    rv[i, pl.ds(128, 16)] = jnp.exp(lr - c16)
                return carry2

            lax.fori_loop(0, _CHUNK // 8, p_body, 0)

            def scale_body(i, carry2):
                p = rv[i, pl.ds(128, 16)]
                loads = [rv[i, pl.ds(h * 16, 16)] for h in range(_H)]
                for h in range(_H):
                    rv[i, pl.ds(h * 16, 16)] = loads[h] * p[h]
                return carry2

            lax.fori_loop(0, _CHUNK, scale_body, 0, unroll=4)

        def do_chunk(j, k):
            k1 = 1 - k
            gather_desc(k).wait()

            @pl.when(j + 1 < _NCHUNK)
            def _():
                for d in idx_descs(j + 1, k1):
                    d.wait()
                gather_desc(k1).start()

            compute(k)
            pltpu.sync_copy(rows_v[k], acc_sh.at[dst_v[k]], add=True)

            @pl.when(j + 2 < _NCHUNK)
            def _():
                for d in idx_descs(j + 2, k):
                    d.start()

        for d in idx_descs(0, 0):
            d.start()
        for d in idx_descs(1, 1):
            d.start()
        for d in idx_descs(0, 0):
            d.wait()
        gather_desc(0).start()

        def pair_body(jj, carry):
            do_chunk(2 * jj, 0)
            do_chunk(2 * jj + 1, 1)
            return carry

        lax.fori_loop(0, _NCHUNK // 2, pair_body, 0)
        if _NCHUNK % 2:
            do_chunk(_NCHUNK - 1, 0)
        plsc.subcore_barrier()

        r0 = s * 624
        pltpu.sync_copy(acc_sh.at[pl.ds(r0, 624)],
                        out_hbm.at[c].at[pl.ds(r0, 624)])

        @pl.when(s == 15)
        def _():
            pltpu.sync_copy(acc_sh.at[pl.ds(9984, 16)],
                            out_hbm.at[c].at[pl.ds(9984, 16)])


def _post_body(p0_ref, p1_ref, s_ref, g_ref, b_ref, w1_ref, b1_ref,
               w2_ref, b2_ref, o_ref):
    P = p0_ref[...] + p1_ref[...]
    hacc = P[:, :128]
    den = P[:, 128:136]
    recip = 1.0 / (den + 1e-10)
    ii = lax.broadcasted_iota(jnp.int32, (_H, 128), 0)
    jj = lax.broadcasted_iota(jnp.int32, (_H, 128), 1)
    expand = jnp.where(ii == jj // 16, 1.0, 0.0).astype(jnp.float32)
    hsum = hacc * jnp.dot(recip, expand, precision=_HP)
    h = jnp.where(hsum > 0, hsum, jnp.exp(jnp.minimum(hsum, 0.0)) - 1.0)
    h = h + s_ref[...]
    mu = jnp.mean(h, axis=-1, keepdims=True)
    var = jnp.mean((h - mu) ** 2, axis=-1, keepdims=True)
    hn = (h - mu) * lax.rsqrt(var + 1e-6) * g_ref[...] + b_ref[...]
    pre = jnp.dot(hn, w1_ref[...], precision=_HP) + b1_ref[...]
    inter = 0.5 * pre * (1.0 + lax.erf(pre * 0.7071067811865476))
    o_ref[...] = jnp.dot(inter, w2_ref[...], precision=_HP) + b2_ref[...] + h


def _full(shape):
    return pl.BlockSpec(shape, lambda i: (0, 0))


def _prea(w, wcat, wa1p):
    grid = _NW // _PREA_BLK
    return pl.pallas_call(
        _prea_body,
        grid=(grid,),
        in_specs=[
            pl.BlockSpec((_PREA_BLK, _IN), lambda i: (i, 0)),
            _full((_IN, _OUT)),
            _full((_IN, 16)),
        ],
        out_specs=[
            pl.BlockSpec((_PREA_BLK, _EXT), lambda i: (i, 0)),
            _full((1, 16)),
        ],
        out_shape=[
            jax.ShapeDtypeStruct((_NW, _EXT), jnp.float32),
            jax.ShapeDtypeStruct((1, 16), jnp.float32),
        ],
    )(w, wcat, wa1p)


def _preb(ea_pk, wb, ba128):
    rows = _E // 8
    blk = 4000
    return pl.pallas_call(
        _preb_body,
        grid=(rows // blk,),
        in_specs=[
            pl.BlockSpec((blk, 128), lambda i: (i, 0)),
            _full((128, 128)),
            _full((1, 128)),
        ],
        out_specs=[
            pl.BlockSpec((blk, 128), lambda i: (i, 0)),
            _full((1, 128)),
        ],
        out_shape=[
            jax.ShapeDtypeStruct((rows, 128), jnp.float32),
            jax.ShapeDtypeStruct((1, 128), jnp.float32),
        ],
    )(ea_pk, wb, ba128)


@functools.cache
def _sc_edge_kernel():
    @functools.partial(
        pl.kernel,
        mesh=plsc.VectorSubcoreMesh(core_axis_name="c", subcore_axis_name="s"),
        out_type=jax.ShapeDtypeStruct((2, _NS, _EXT), jnp.float32),
        compiler_params=pltpu.CompilerParams(use_tc_tiling_on_sc=False),
        scratch_types=[
            pltpu.VMEM((_CHUNK,), jnp.int32),
            pltpu.VMEM((_CHUNK,), jnp.int32),
            pltpu.VMEM((_CHUNK,), jnp.int32),
            pltpu.VMEM((_CHUNK,), jnp.int32),
            pltpu.VMEM((_CHUNK // 8, 128), jnp.float32),
            pltpu.VMEM((_CHUNK // 8, 128), jnp.float32),
            pltpu.VMEM((_CHUNK, _EXT), jnp.float32),
            pltpu.VMEM((_CHUNK, _EXT), jnp.float32),
            pltpu.VMEM((16,), jnp.float32),
            pltpu.VMEM((128,), jnp.float32),
            pltpu.VMEM_SHARED((_NS, _EXT), jnp.float32),
            pltpu.SemaphoreType.DMA,
            pltpu.SemaphoreType.DMA,
            pltpu.SemaphoreType.DMA,
            pltpu.SemaphoreType.DMA,
        ],
    )
    def _sc_edge(*refs):
        _sc_edge_body(*refs)

    return _sc_edge


def _post(p0, p1, s, ln_g, ln_b, w1, b1, w2, b2):
    grid = _NS // _POST_BLK
    return pl.pallas_call(
        _post_body,
        grid=(grid,),
        in_specs=[
            pl.BlockSpec((_POST_BLK, _EXT), lambda i: (i, 0)),
            pl.BlockSpec((_POST_BLK, _EXT), lambda i: (i, 0)),
            pl.BlockSpec((_POST_BLK, _OUT), lambda i: (i, 0)),
            _full((1, _OUT)),
            _full((1, _OUT)),
            _full((_OUT, _FFN)),
            _full((1, _FFN)),
            _full((_FFN, _OUT)),
            _full((1, _OUT)),
        ],
        out_specs=pl.BlockSpec((_POST_BLK, _OUT), lambda i: (i, 0)),
        out_shape=jax.ShapeDtypeStruct((_NS, _OUT), jnp.float32),
    )(p0, p1, s, ln_g, ln_b, w1, b1, w2, b2)


def kernel(w, s, edge_index, edge_attr, W_fc, attn_a, W_feat, b_feat,
           ln_g, ln_b, w1, b1, w2, b2):
    src = edge_index[0]
    dst = edge_index[1]
    a1 = attn_a[:, :_DH]
    a2 = attn_a[:, _DH:]
    wcat = jnp.transpose(W_fc, (1, 0, 2)).reshape(_IN, _OUT)
    wa1p = jnp.pad(jnp.einsum('hdo,ho->dh', W_fc, a1), ((0, 0), (0, 8)))
    wa2p = jnp.pad(jnp.einsum('hdo,ho->dh', W_feat, a2), ((0, 0), (0, 8)))
    bap = jnp.pad(jnp.einsum('ho,ho->h', b_feat, a2), (0, 8))
    wb = jnp.kron(jnp.eye(8, dtype=jnp.float32), wa2p)
    ba128 = jnp.tile(bap, 8).reshape(1, 128)
    ea_pk = edge_attr.reshape(_E // 8, 128)

    z2ext, zamax = _prea(w, wcat, wa1p)
    eapk, eamax = _preb(ea_pk, wb, ba128)

    zeros = jnp.zeros((_NS, _EXT), jnp.float32)
    partials = _sc_edge_kernel()(z2ext, src, dst, eapk,
                                 zamax.reshape(16), eamax.reshape(128), zeros)

    return _post(partials[0], partials[1], s,
                 ln_g.reshape(1, _OUT), ln_b.reshape(1, _OUT),
                 w1, b1.reshape(1, _FFN), w2, b2.reshape(1, _OUT))
